# baseline (speedup 1.0000x reference)
import functools

import jax
import jax.numpy as jnp
from jax import lax
from jax.experimental import pallas as pl
from jax.experimental.pallas import tpu as pltpu
from jax.experimental.pallas import tpu_sc as plsc

NUM_REL = 32
DIM = 64
NN = 16


def _sc_fused(item_ids, usr_id, adj_ent, adj_rel, ent_table, usr_table, rel_table):
    B = item_ids.shape[0]
    info = plsc.get_sparse_core_info()
    NC, NS = info.num_cores, info.num_subcores
    NW = NC * NS
    bpw = B // NW

    mesh = plsc.VectorSubcoreMesh(core_axis_name="c", subcore_axis_name="s")

    out_type = (
        jax.ShapeDtypeStruct((B, DIM), jnp.float32),
        jax.ShapeDtypeStruct((B, DIM), jnp.float32),
        jax.ShapeDtypeStruct((B, DIM), jnp.float32),
        jax.ShapeDtypeStruct((B, NN, DIM), jnp.float32),
        jax.ShapeDtypeStruct((B, NN), jnp.float32),
    )
    f32, i32 = jnp.float32, jnp.int32
    scratch = [
        pltpu.VMEM((bpw,), i32),
        pltpu.VMEM((bpw,), i32),
        pltpu.VMEM((bpw, NN), i32),
        pltpu.VMEM((bpw, NN), i32),
        pltpu.VMEM((bpw, DIM), f32),
        pltpu.VMEM((bpw, DIM), f32),
        pltpu.VMEM((NUM_REL, DIM), f32),
        pltpu.VMEM((NUM_REL, bpw), f32),
        pltpu.VMEM((NN,), f32),
        [pltpu.VMEM((NN, NN), i32)] * 2,
        [pltpu.VMEM((NN, NN), i32)] * 2,
        [pltpu.VMEM((2, 128), i32)] * 2,
        [pltpu.VMEM((NN, DIM), f32)] * 2,
        [pltpu.VMEM((NN * NN, DIM), f32)] * 2,
        [pltpu.VMEM((NN,), f32)] * 2,
        [pltpu.VMEM((DIM,), f32)] * 2,
        [pltpu.VMEM((NN, DIM), f32)] * 2,
        pltpu.SemaphoreType.DMA,
        [pltpu.SemaphoreType.DMA] * 2,
        [pltpu.SemaphoreType.DMA] * 2,
        [pltpu.SemaphoreType.DMA] * 2,
        [pltpu.SemaphoreType.DMA] * 2,
    ]

    @functools.partial(pl.kernel, out_type=out_type, mesh=mesh,
                       scratch_types=scratch,
                       compiler_params=pltpu.CompilerParams(
                           use_tc_tiling_on_sc=False,
                           needs_layout_passes=False))
    def k(item_h, usr_h, adj_ent_h, adj_rel_h, ent_h, usrt_h, rel_h,
          u_o, e0_o, s0_o, agg1_o, w1_o,
          it_v, us_v, a1_v, r1_v, e0_v, u_v, rel_v, p_v, ebuf,
          a2_s, r2_s, flat_s, e1_s, e2_s, w1buf_s, s0buf_s, aggbuf_s,
          sem_hdr, sem_a2, sem_er, sem_e2, sem_out):
        wid = lax.axis_index("s") * NC + lax.axis_index("c")
        base = wid * bpw
        iota16 = lax.iota(i32, NN)
        zeros16 = jnp.zeros((NN,), f32)

        pltpu.sync_copy(item_h.at[pl.ds(base, bpw)], it_v)
        pltpu.sync_copy(usr_h.at[pl.ds(base, bpw)], us_v)
        h1 = pltpu.async_copy(adj_ent_h.at[it_v], a1_v, sem_hdr)
        h2 = pltpu.async_copy(adj_rel_h.at[it_v], r1_v, sem_hdr)
        h3 = pltpu.async_copy(ent_h.at[it_v], e0_v, sem_hdr)
        h4 = pltpu.async_copy(usrt_h.at[us_v], u_v, sem_hdr)
        pltpu.sync_copy(rel_h, rel_v)
        h1.wait(); h2.wait(); h3.wait(); h4.wait()
        pltpu.sync_copy(e0_v, e0_o.at[pl.ds(base, bpw)])
        pltpu.sync_copy(u_v, u_o.at[pl.ds(base, bpw)])

        for kk in range(NUM_REL):
            p_v[kk, pl.ds(0, NN)] = zeros16
            p_v[kk, pl.ds(NN, NN)] = zeros16

        def pbody(d, carry):
            dvec = jnp.full((NN,), d, i32)
            ucol0 = plsc.load_gather(u_v, [iota16, dvec])
            ucol1 = plsc.load_gather(u_v, [iota16 + NN, dvec])
            for kk in range(NUM_REL):
                wvec = plsc.load_gather(rel_v, [jnp.full((NN,), kk, i32), dvec])
                plsc.addupdate(p_v.at[kk, pl.ds(0, NN)], ucol0 * wvec)
                plsc.addupdate(p_v.at[kk, pl.ds(NN, NN)], ucol1 * wvec)
            return carry

        lax.fori_loop(0, DIM, pbody, 0)

        def fire_a2(i, s):
            return pltpu.async_copy(adj_ent_h.at[a1_v.at[i]], a2_s[s], sem_a2[s])

        def fire_er(i, s):
            pltpu.async_copy(ent_h.at[a1_v.at[i]], e1_s[s], sem_er[s])
            pltpu.async_copy(adj_rel_h.at[a1_v.at[i]], r2_s[s], sem_er[s])

        def wait_a2(i, s):
            pltpu.make_async_copy(adj_ent_h.at[a1_v.at[i]], a2_s[s], sem_a2[s]).wait()

        def flatten_fire_e2(i, s):
            for j in range(NN):
                flat_s[s][j // 8, pl.ds((j % 8) * NN, NN)] = a2_s[s][j, :]
            pltpu.async_copy(ent_h.at[flat_s[s].at[0]],
                             e2_s[s].at[pl.ds(0, 128)], sem_e2[s])
            pltpu.async_copy(ent_h.at[flat_s[s].at[1]],
                             e2_s[s].at[pl.ds(128, 128)], sem_e2[s])

        def drain_outs(i, s):
            gbp = base + i
            pltpu.make_async_copy(w1buf_s[s], w1_o.at[gbp], sem_out[s]).wait()
            pltpu.make_async_copy(s0buf_s[s], s0_o.at[gbp], sem_out[s]).wait()
            pltpu.make_async_copy(aggbuf_s[s], agg1_o.at[gbp], sem_out[s]).wait()

        def compute(i, s):
            gb = base + i

            @pl.when(i >= 2)
            def _():
                drain_outs(i - 2, s)

            pltpu.make_async_copy(ent_h.at[a1_v.at[i]], e1_s[s], sem_er[s]).wait()
            pltpu.make_async_copy(adj_rel_h.at[a1_v.at[i]], r2_s[s], sem_er[s]).wait()
            pltpu.make_async_copy(ent_h.at[flat_s[s].at[0]],
                                  e2_s[s].at[pl.ds(0, 128)], sem_e2[s]).wait()
            pltpu.make_async_copy(ent_h.at[flat_s[s].at[1]],
                                  e2_s[s].at[pl.ds(128, 128)], sem_e2[s]).wait()

            ivec = jnp.full((NN,), i, i32)
            r1vec = plsc.load_gather(r1_v, [ivec, iota16])
            sc1 = plsc.load_gather(p_v, [r1vec, ivec])
            es1 = jnp.exp(sc1)
            w1vec = es1 / jnp.sum(es1)
            w1buf_s[s][...] = w1vec
            acc0 = [zeros16] * 4
            for n in range(NN):
                w = plsc.load_gather(w1buf_s[s], [jnp.full((NN,), n, i32)])
                for c in range(4):
                    acc0[c] = acc0[c] + w * e1_s[s][n, pl.ds(16 * c, 16)]
            for c in range(4):
                s0buf_s[s][pl.ds(16 * c, 16)] = acc0[c]

            wn = []
            esum = zeros16
            for n in range(NN):
                r2col = plsc.load_gather(r2_s[s], [iota16, jnp.full((NN,), n, i32)])
                en = jnp.exp(plsc.load_gather(p_v, [r2col, ivec]))
                wn.append(en)
                esum = esum + en
            wn = [en / esum for en in wn]
            col_n = [iota16 * NN + n for n in range(NN)]

            def dbody(dc, carry):
                for q in range(4):
                    d = dc * 4 + q
                    dvec = jnp.full((NN,), d, i32)
                    accd = plsc.load_gather(e1_s[s], [iota16, dvec])
                    for n in range(NN):
                        e2col = plsc.load_gather(e2_s[s], [col_n[n], dvec])
                        accd = accd + wn[n] * e2col
                    plsc.store_scatter(aggbuf_s[s], [iota16, dvec], accd)
                return carry

            lax.fori_loop(0, DIM // 4, dbody, 0)

            pltpu.async_copy(w1buf_s[s], w1_o.at[gb], sem_out[s])
            pltpu.async_copy(s0buf_s[s], s0_o.at[gb], sem_out[s])
            pltpu.async_copy(aggbuf_s[s], agg1_o.at[gb], sem_out[s])

        fire_a2(0, 0)
        fire_er(0, 0)
        wait_a2(0, 0)
        flatten_fire_e2(0, 0)
        fire_a2(1, 1)
        fire_er(1, 1)

        def body(j, carry):
            r0 = 2 * j
            more = j < (bpw // 2 - 1)

            @pl.when(more)
            def _():
                fire_a2(r0 + 2, 0)

            wait_a2(r0 + 1, 1)
            flatten_fire_e2(r0 + 1, 1)
            compute(r0, 0)

            @pl.when(more)
            def _():
                fire_er(r0 + 2, 0)
                wait_a2(r0 + 2, 0)
                flatten_fire_e2(r0 + 2, 0)
                fire_a2(r0 + 3, 1)

            compute(r0 + 1, 1)

            @pl.when(more)
            def _():
                fire_er(r0 + 3, 1)

            return carry

        lax.fori_loop(0, bpw // 2, body, 0)

        drain_outs(bpw - 2, 0)
        drain_outs(bpw - 1, 1)

    return k(item_ids, usr_id, adj_ent, adj_rel, ent_table, usr_table, rel_table)


def _tc_body(u_r, e0_r, s0_r, agg_r, w1_r, W_r, b_r, out_r):
    B = u_r.shape[0]
    Wt = W_r[...].T
    bb = b_r[...]
    h0 = jax.nn.sigmoid(
        jnp.dot(e0_r[...] + s0_r[...], Wt, preferred_element_type=jnp.float32) + bb)
    agg = agg_r[...]
    h1 = jax.nn.sigmoid(
        (jnp.dot(agg.reshape(B * NN, DIM), Wt,
                 preferred_element_type=jnp.float32) + bb).reshape(B, NN, DIM))
    w1 = w1_r[...]
    acc = jnp.zeros((B, DIM), jnp.float32)
    for n in range(NN):
        acc = acc + w1[:, n][:, None] * h1[:, n, :]
    f = jnp.tanh(jnp.dot(h0 + acc, Wt, preferred_element_type=jnp.float32) + bb)
    out_r[...] = jax.nn.sigmoid(jnp.sum(u_r[...] * f, axis=-1))


def _tc_dense(u, e0, s0, agg1, w1, W, b):
    B = u.shape[0]
    return pl.pallas_call(
        _tc_body,
        out_shape=jax.ShapeDtypeStruct((B,), jnp.float32),
        compiler_params=pltpu.CompilerParams(
            vmem_limit_bytes=100 * 1024 * 1024),
    )(u, e0, s0, agg1, w1, W, b)


def kernel(usr_id, item_ids, adj_ent, adj_rel, usr_table, ent_table, rel_table, W, b):
    B = usr_id.shape[0]
    item_flat = item_ids.reshape(B).astype(jnp.int32)
    usr_flat = usr_id.reshape(B).astype(jnp.int32)
    adj_ent = adj_ent.astype(jnp.int32)
    adj_rel = adj_rel.astype(jnp.int32)

    u, e0, s0, agg1, w1 = _sc_fused(
        item_flat, usr_flat, adj_ent, adj_rel, ent_table, usr_table, rel_table)

    return _tc_dense(u, e0, s0, agg1, w1, W, b.reshape(1, DIM))


# speedup vs baseline: 1.8113x; 1.8113x over previous
import functools

import jax
import jax.numpy as jnp
from jax import lax
from jax.experimental import pallas as pl
from jax.experimental.pallas import tpu as pltpu
from jax.experimental.pallas import tpu_sc as plsc

NUM_REL = 32
DIM = 64
NN = 16


def _sc_fused(item_ids, usr_id, adj_ent, adj_rel, ent_table, usr_table, rel_table):
    B = item_ids.shape[0]
    info = plsc.get_sparse_core_info()
    NC, NS = info.num_cores, info.num_subcores
    NW = NC * NS
    bpw = B // NW

    mesh = plsc.VectorSubcoreMesh(core_axis_name="c", subcore_axis_name="s")

    out_type = (
        jax.ShapeDtypeStruct((B, DIM), jnp.float32),
        jax.ShapeDtypeStruct((B, DIM), jnp.float32),
        jax.ShapeDtypeStruct((B, DIM), jnp.float32),
        jax.ShapeDtypeStruct((B, NN, DIM), jnp.float32),
        jax.ShapeDtypeStruct((B, NN), jnp.float32),
    )
    f32, i32 = jnp.float32, jnp.int32
    scratch = [
        pltpu.VMEM((bpw,), i32),
        pltpu.VMEM((bpw,), i32),
        pltpu.VMEM((bpw, NN), i32),
        pltpu.VMEM((bpw, NN), i32),
        pltpu.VMEM((bpw, DIM), f32),
        pltpu.VMEM((bpw, DIM), f32),
        pltpu.VMEM((NUM_REL, DIM), f32),
        pltpu.VMEM((NUM_REL, bpw), f32),
        pltpu.VMEM((NN,), f32),
        [pltpu.VMEM((NN, NN), i32)] * 2,
        [pltpu.VMEM((NN, NN), i32)] * 2,
        [pltpu.VMEM((2, 128), i32)] * 2,
        [pltpu.VMEM((NN, DIM), f32)] * 2,
        [pltpu.VMEM((NN * NN, DIM), f32)] * 2,
        [pltpu.VMEM((NN,), f32)] * 2,
        [pltpu.VMEM((DIM,), f32)] * 2,
        [pltpu.VMEM((NN, DIM), f32)] * 2,
        pltpu.SemaphoreType.DMA,
        [pltpu.SemaphoreType.DMA] * 2,
        [pltpu.SemaphoreType.DMA] * 2,
        [pltpu.SemaphoreType.DMA] * 2,
        [pltpu.SemaphoreType.DMA] * 2,
    ]

    @functools.partial(pl.kernel, out_type=out_type, mesh=mesh,
                       scratch_types=scratch,
                       compiler_params=pltpu.CompilerParams(
                           use_tc_tiling_on_sc=False,
                           needs_layout_passes=False))
    def k(item_h, usr_h, adj_ent_h, adj_rel_h, ent_h, usrt_h, rel_h,
          u_o, e0_o, s0_o, agg1_o, w1_o,
          it_v, us_v, a1_v, r1_v, e0_v, u_v, rel_v, p_v, ebuf,
          a2_s, r2_s, flat_s, e1_s, e2_s, w1buf_s, s0buf_s, aggbuf_s,
          sem_hdr, sem_a2, sem_er, sem_e2, sem_out):
        wid = lax.axis_index("s") * NC + lax.axis_index("c")
        base = wid * bpw
        iota16 = lax.iota(i32, NN)
        zeros16 = jnp.zeros((NN,), f32)

        pltpu.sync_copy(item_h.at[pl.ds(base, bpw)], it_v)
        pltpu.sync_copy(usr_h.at[pl.ds(base, bpw)], us_v)
        h1 = pltpu.async_copy(adj_ent_h.at[it_v], a1_v, sem_hdr)
        h2 = pltpu.async_copy(adj_rel_h.at[it_v], r1_v, sem_hdr)
        h3 = pltpu.async_copy(ent_h.at[it_v], e0_v, sem_hdr)
        h4 = pltpu.async_copy(usrt_h.at[us_v], u_v, sem_hdr)
        pltpu.sync_copy(rel_h, rel_v)
        h1.wait(); h2.wait(); h3.wait(); h4.wait()
        pltpu.sync_copy(e0_v, e0_o.at[pl.ds(base, bpw)])
        pltpu.sync_copy(u_v, u_o.at[pl.ds(base, bpw)])

        for kk in range(NUM_REL):
            p_v[kk, pl.ds(0, NN)] = zeros16
            p_v[kk, pl.ds(NN, NN)] = zeros16

        def pbody(d, carry):
            dvec = jnp.full((NN,), d, i32)
            ucol0 = plsc.load_gather(u_v, [iota16, dvec])
            ucol1 = plsc.load_gather(u_v, [iota16 + NN, dvec])
            for kk in range(NUM_REL):
                wvec = plsc.load_gather(rel_v, [jnp.full((NN,), kk, i32), dvec])
                plsc.addupdate(p_v.at[kk, pl.ds(0, NN)], ucol0 * wvec)
                plsc.addupdate(p_v.at[kk, pl.ds(NN, NN)], ucol1 * wvec)
            return carry

        lax.fori_loop(0, DIM, pbody, 0)

        def fire_a2(i, s):
            return pltpu.async_copy(adj_ent_h.at[a1_v.at[i]], a2_s[s], sem_a2[s])

        def fire_er(i, s):
            pltpu.async_copy(ent_h.at[a1_v.at[i]], e1_s[s], sem_er[s])
            pltpu.async_copy(adj_rel_h.at[a1_v.at[i]], r2_s[s], sem_er[s])

        def wait_a2(i, s):
            pltpu.make_async_copy(adj_ent_h.at[a1_v.at[i]], a2_s[s], sem_a2[s]).wait()

        def flatten_fire_e2(i, s):
            for j in range(NN):
                flat_s[s][j // 8, pl.ds((j % 8) * NN, NN)] = a2_s[s][j, :]
            pltpu.async_copy(ent_h.at[flat_s[s].at[0]],
                             e2_s[s].at[pl.ds(0, 128)], sem_e2[s])
            pltpu.async_copy(ent_h.at[flat_s[s].at[1]],
                             e2_s[s].at[pl.ds(128, 128)], sem_e2[s])

        def drain_outs(i, s):
            gbp = base + i
            pltpu.make_async_copy(w1buf_s[s], w1_o.at[gbp], sem_out[s]).wait()
            pltpu.make_async_copy(s0buf_s[s], s0_o.at[gbp], sem_out[s]).wait()
            pltpu.make_async_copy(aggbuf_s[s], agg1_o.at[gbp], sem_out[s]).wait()

        def compute(i, s):
            gb = base + i

            @pl.when(i >= 2)
            def _():
                drain_outs(i - 2, s)

            pltpu.make_async_copy(ent_h.at[a1_v.at[i]], e1_s[s], sem_er[s]).wait()
            pltpu.make_async_copy(adj_rel_h.at[a1_v.at[i]], r2_s[s], sem_er[s]).wait()
            pltpu.make_async_copy(ent_h.at[flat_s[s].at[0]],
                                  e2_s[s].at[pl.ds(0, 128)], sem_e2[s]).wait()
            pltpu.make_async_copy(ent_h.at[flat_s[s].at[1]],
                                  e2_s[s].at[pl.ds(128, 128)], sem_e2[s]).wait()

            ivec = jnp.full((NN,), i, i32)
            r1vec = plsc.load_gather(r1_v, [ivec, iota16])
            sc1 = plsc.load_gather(p_v, [r1vec, ivec])
            es1 = jnp.exp(sc1)
            w1vec = es1 / jnp.sum(es1)
            w1buf_s[s][...] = w1vec
            acc0 = [zeros16] * 4
            for n in range(NN):
                w = plsc.load_gather(w1buf_s[s], [jnp.full((NN,), n, i32)])
                for c in range(4):
                    acc0[c] = acc0[c] + w * e1_s[s][n, pl.ds(16 * c, 16)]
            for c in range(4):
                s0buf_s[s][pl.ds(16 * c, 16)] = acc0[c]

            for m in range(NN):
                r2vec = r2_s[s][m, :]
                sc2 = plsc.load_gather(p_v, [r2vec, ivec])
                es2 = jnp.exp(sc2)
                ssum2 = jnp.sum(es2)
                ebuf[...] = es2
                acca = [zeros16] * 4
                accb = [zeros16] * 4
                for n in range(0, NN, 2):
                    wa = plsc.load_gather(ebuf, [jnp.full((NN,), n, i32)])
                    wb = plsc.load_gather(ebuf, [jnp.full((NN,), n + 1, i32)])
                    rowa = m * NN + n
                    rowb = rowa + 1
                    for c in range(4):
                        acca[c] = acca[c] + wa * e2_s[s][rowa, pl.ds(16 * c, 16)]
                        accb[c] = accb[c] + wb * e2_s[s][rowb, pl.ds(16 * c, 16)]
                for c in range(4):
                    aggbuf_s[s][m, pl.ds(16 * c, 16)] = (
                        e1_s[s][m, pl.ds(16 * c, 16)] + (acca[c] + accb[c]) / ssum2)

            pltpu.async_copy(w1buf_s[s], w1_o.at[gb], sem_out[s])
            pltpu.async_copy(s0buf_s[s], s0_o.at[gb], sem_out[s])
            pltpu.async_copy(aggbuf_s[s], agg1_o.at[gb], sem_out[s])

        fire_a2(0, 0)
        fire_er(0, 0)
        wait_a2(0, 0)
        flatten_fire_e2(0, 0)
        fire_a2(1, 1)
        fire_er(1, 1)

        def body(j, carry):
            r0 = 2 * j
            more = j < (bpw // 2 - 1)

            @pl.when(more)
            def _():
                fire_a2(r0 + 2, 0)

            wait_a2(r0 + 1, 1)
            flatten_fire_e2(r0 + 1, 1)
            compute(r0, 0)

            @pl.when(more)
            def _():
                fire_er(r0 + 2, 0)
                wait_a2(r0 + 2, 0)
                flatten_fire_e2(r0 + 2, 0)
                fire_a2(r0 + 3, 1)

            compute(r0 + 1, 1)

            @pl.when(more)
            def _():
                fire_er(r0 + 3, 1)

            return carry

        lax.fori_loop(0, bpw // 2, body, 0)

        drain_outs(bpw - 2, 0)
        drain_outs(bpw - 1, 1)

    return k(item_ids, usr_id, adj_ent, adj_rel, ent_table, usr_table, rel_table)


def _tc_body(u_r, e0_r, s0_r, agg_r, w1_r, W_r, b_r, out_r):
    B = u_r.shape[0]
    Wt = W_r[...].T
    bb = b_r[...]
    h0 = jax.nn.sigmoid(
        jnp.dot(e0_r[...] + s0_r[...], Wt, preferred_element_type=jnp.float32) + bb)
    agg = agg_r[...]
    h1 = jax.nn.sigmoid(
        (jnp.dot(agg.reshape(B * NN, DIM), Wt,
                 preferred_element_type=jnp.float32) + bb).reshape(B, NN, DIM))
    w1 = w1_r[...]
    acc = jnp.zeros((B, DIM), jnp.float32)
    for n in range(NN):
        acc = acc + w1[:, n][:, None] * h1[:, n, :]
    f = jnp.tanh(jnp.dot(h0 + acc, Wt, preferred_element_type=jnp.float32) + bb)
    out_r[...] = jax.nn.sigmoid(jnp.sum(u_r[...] * f, axis=-1))


def _tc_dense(u, e0, s0, agg1, w1, W, b):
    B = u.shape[0]
    return pl.pallas_call(
        _tc_body,
        out_shape=jax.ShapeDtypeStruct((B,), jnp.float32),
        compiler_params=pltpu.CompilerParams(
            vmem_limit_bytes=100 * 1024 * 1024),
    )(u, e0, s0, agg1, w1, W, b)


def kernel(usr_id, item_ids, adj_ent, adj_rel, usr_table, ent_table, rel_table, W, b):
    B = usr_id.shape[0]
    item_flat = item_ids.reshape(B).astype(jnp.int32)
    usr_flat = usr_id.reshape(B).astype(jnp.int32)
    adj_ent = adj_ent.astype(jnp.int32)
    adj_rel = adj_rel.astype(jnp.int32)

    u, e0, s0, agg1, w1 = _sc_fused(
        item_flat, usr_flat, adj_ent, adj_rel, ent_table, usr_table, rel_table)

    return _tc_dense(u, e0, s0, agg1, w1, W, b.reshape(1, DIM))


# speedup vs baseline: 1.9521x; 1.0777x over previous
import functools

import jax
import jax.numpy as jnp
from jax import lax
from jax.experimental import pallas as pl
from jax.experimental.pallas import tpu as pltpu
from jax.experimental.pallas import tpu_sc as plsc

NUM_REL = 32
DIM = 64
NN = 16


def _sc_fused(item_ids, usr_id, adj_cat, ent_table, usr_table, rel_table):
    B = item_ids.shape[0]
    info = plsc.get_sparse_core_info()
    NC, NS = info.num_cores, info.num_subcores
    NW = NC * NS
    bpw = B // NW

    mesh = plsc.VectorSubcoreMesh(core_axis_name="c", subcore_axis_name="s")

    out_type = (
        jax.ShapeDtypeStruct((B, DIM), jnp.float32),
        jax.ShapeDtypeStruct((B, DIM), jnp.float32),
        jax.ShapeDtypeStruct((B, DIM), jnp.float32),
        jax.ShapeDtypeStruct((B, NN, DIM), jnp.float32),
        jax.ShapeDtypeStruct((B, NN), jnp.float32),
    )
    f32, i32 = jnp.float32, jnp.int32
    scratch = [
        pltpu.VMEM((bpw,), i32),
        pltpu.VMEM((bpw,), i32),
        pltpu.VMEM((bpw, 2 * NN), i32),
        pltpu.VMEM((bpw, DIM), f32),
        pltpu.VMEM((bpw, DIM), f32),
        pltpu.VMEM((NUM_REL, DIM), f32),
        pltpu.VMEM((NUM_REL, bpw), f32),
        pltpu.VMEM((NN,), f32),
        [pltpu.VMEM((NN, 2 * NN), i32)] * 2,
        [pltpu.VMEM((2, 128), i32)] * 2,
        [pltpu.VMEM((NN, DIM), f32)] * 2,
        [pltpu.VMEM((NN * NN, DIM), f32)] * 2,
        [pltpu.VMEM((NN,), f32)] * 2,
        [pltpu.VMEM((DIM,), f32)] * 2,
        [pltpu.VMEM((NN, DIM), f32)] * 2,
        pltpu.SemaphoreType.DMA,
        [pltpu.SemaphoreType.DMA] * 2,
        [pltpu.SemaphoreType.DMA] * 2,
        [pltpu.SemaphoreType.DMA] * 2,
        [pltpu.SemaphoreType.DMA] * 2,
    ]

    @functools.partial(pl.kernel, out_type=out_type, mesh=mesh,
                       scratch_types=scratch,
                       compiler_params=pltpu.CompilerParams(
                           use_tc_tiling_on_sc=False,
                           needs_layout_passes=False))
    def k(item_h, usr_h, adj_h, ent_h, usrt_h, rel_h,
          u_o, e0_o, s0_o, agg1_o, w1_o,
          it_v, us_v, ar1_v, e0_v, u_v, rel_v, p_v, ebuf,
          ar2_s, flat_s, e1_s, e2_s, w1buf_s, s0buf_s, aggbuf_s,
          sem_hdr, sem_a2, sem_er, sem_e2, sem_out):
        wid = lax.axis_index("s") * NC + lax.axis_index("c")
        base = wid * bpw
        iota16 = lax.iota(i32, NN)
        zeros16 = jnp.zeros((NN,), f32)

        pltpu.sync_copy(item_h.at[pl.ds(base, bpw)], it_v)
        pltpu.sync_copy(usr_h.at[pl.ds(base, bpw)], us_v)
        h1 = pltpu.async_copy(adj_h.at[it_v], ar1_v, sem_hdr)
        h3 = pltpu.async_copy(ent_h.at[it_v], e0_v, sem_hdr)
        h4 = pltpu.async_copy(usrt_h.at[us_v], u_v, sem_hdr)
        pltpu.sync_copy(rel_h, rel_v)
        h1.wait(); h3.wait(); h4.wait()
        pltpu.sync_copy(e0_v, e0_o.at[pl.ds(base, bpw)])
        pltpu.sync_copy(u_v, u_o.at[pl.ds(base, bpw)])

        for kk in range(NUM_REL):
            p_v[kk, pl.ds(0, NN)] = zeros16
            p_v[kk, pl.ds(NN, NN)] = zeros16

        def pbody(d, carry):
            dvec = jnp.full((NN,), d, i32)
            ucol0 = plsc.load_gather(u_v, [iota16, dvec])
            ucol1 = plsc.load_gather(u_v, [iota16 + NN, dvec])
            for kk in range(NUM_REL):
                wvec = plsc.load_gather(rel_v, [jnp.full((NN,), kk, i32), dvec])
                plsc.addupdate(p_v.at[kk, pl.ds(0, NN)], ucol0 * wvec)
                plsc.addupdate(p_v.at[kk, pl.ds(NN, NN)], ucol1 * wvec)
            return carry

        lax.fori_loop(0, DIM, pbody, 0)

        def fire_a2(i, s):
            return pltpu.async_copy(adj_h.at[ar1_v.at[i, pl.ds(0, NN)]],
                                    ar2_s[s], sem_a2[s])

        def fire_er(i, s):
            pltpu.async_copy(ent_h.at[ar1_v.at[i, pl.ds(0, NN)]], e1_s[s], sem_er[s])

        def wait_a2(i, s):
            pltpu.make_async_copy(adj_h.at[ar1_v.at[i, pl.ds(0, NN)]],
                                  ar2_s[s], sem_a2[s]).wait()

        def flatten_fire_e2(i, s):
            for j in range(NN):
                flat_s[s][j // 8, pl.ds((j % 8) * NN, NN)] = ar2_s[s][j, pl.ds(0, NN)]
            pltpu.async_copy(ent_h.at[flat_s[s].at[0]],
                             e2_s[s].at[pl.ds(0, 128)], sem_e2[s])
            pltpu.async_copy(ent_h.at[flat_s[s].at[1]],
                             e2_s[s].at[pl.ds(128, 128)], sem_e2[s])

        def drain_outs(i, s):
            gbp = base + i
            pltpu.make_async_copy(w1buf_s[s], w1_o.at[gbp], sem_out[s]).wait()
            pltpu.make_async_copy(s0buf_s[s], s0_o.at[gbp], sem_out[s]).wait()
            pltpu.make_async_copy(aggbuf_s[s], agg1_o.at[gbp], sem_out[s]).wait()

        def compute(i, s):
            gb = base + i

            @pl.when(i >= 2)
            def _():
                drain_outs(i - 2, s)

            pltpu.make_async_copy(ent_h.at[ar1_v.at[i, pl.ds(0, NN)]],
                                  e1_s[s], sem_er[s]).wait()
            pltpu.make_async_copy(ent_h.at[flat_s[s].at[0]],
                                  e2_s[s].at[pl.ds(0, 128)], sem_e2[s]).wait()
            pltpu.make_async_copy(ent_h.at[flat_s[s].at[1]],
                                  e2_s[s].at[pl.ds(128, 128)], sem_e2[s]).wait()

            ivec = jnp.full((NN,), i, i32)
            r1vec = plsc.load_gather(ar1_v, [ivec, iota16 + NN])
            sc1 = plsc.load_gather(p_v, [r1vec, ivec])
            es1 = jnp.exp(sc1)
            w1vec = es1 / jnp.sum(es1)
            w1buf_s[s][...] = w1vec
            acc0 = [zeros16] * 4
            for n in range(NN):
                w = plsc.load_gather(w1buf_s[s], [jnp.full((NN,), n, i32)])
                for c in range(4):
                    acc0[c] = acc0[c] + w * e1_s[s][n, pl.ds(16 * c, 16)]
            for c in range(4):
                s0buf_s[s][pl.ds(16 * c, 16)] = acc0[c]

            for m in range(NN):
                r2vec = ar2_s[s][m, pl.ds(NN, NN)]
                sc2 = plsc.load_gather(p_v, [r2vec, ivec])
                es2 = jnp.exp(sc2)
                ssum2 = jnp.sum(es2)
                ebuf[...] = es2
                acca = [zeros16] * 4
                accb = [zeros16] * 4
                for n in range(0, NN, 2):
                    wa = plsc.load_gather(ebuf, [jnp.full((NN,), n, i32)])
                    wb = plsc.load_gather(ebuf, [jnp.full((NN,), n + 1, i32)])
                    rowa = m * NN + n
                    rowb = rowa + 1
                    for c in range(4):
                        acca[c] = acca[c] + wa * e2_s[s][rowa, pl.ds(16 * c, 16)]
                        accb[c] = accb[c] + wb * e2_s[s][rowb, pl.ds(16 * c, 16)]
                for c in range(4):
                    aggbuf_s[s][m, pl.ds(16 * c, 16)] = (
                        e1_s[s][m, pl.ds(16 * c, 16)] + (acca[c] + accb[c]) / ssum2)

            pltpu.async_copy(w1buf_s[s], w1_o.at[gb], sem_out[s])
            pltpu.async_copy(s0buf_s[s], s0_o.at[gb], sem_out[s])
            pltpu.async_copy(aggbuf_s[s], agg1_o.at[gb], sem_out[s])

        fire_a2(0, 0)
        fire_er(0, 0)
        wait_a2(0, 0)
        flatten_fire_e2(0, 0)
        fire_a2(1, 1)
        fire_er(1, 1)

        def body(j, carry):
            r0 = 2 * j
            more = j < (bpw // 2 - 1)

            @pl.when(more)
            def _():
                fire_a2(r0 + 2, 0)

            wait_a2(r0 + 1, 1)
            flatten_fire_e2(r0 + 1, 1)
            compute(r0, 0)

            @pl.when(more)
            def _():
                fire_er(r0 + 2, 0)
                wait_a2(r0 + 2, 0)
                flatten_fire_e2(r0 + 2, 0)
                fire_a2(r0 + 3, 1)

            compute(r0 + 1, 1)

            @pl.when(more)
            def _():
                fire_er(r0 + 3, 1)

            return carry

        lax.fori_loop(0, bpw // 2, body, 0)

        drain_outs(bpw - 2, 0)
        drain_outs(bpw - 1, 1)

    return k(item_ids, usr_id, adj_cat, ent_table, usr_table, rel_table)


def _tc_body(u_r, e0_r, s0_r, agg_r, w1_r, W_r, b_r, out_r):
    B = u_r.shape[0]
    Wt = W_r[...].T
    bb = b_r[...]
    h0 = jax.nn.sigmoid(
        jnp.dot(e0_r[...] + s0_r[...], Wt, preferred_element_type=jnp.float32) + bb)
    agg = agg_r[...]
    h1 = jax.nn.sigmoid(
        (jnp.dot(agg.reshape(B * NN, DIM), Wt,
                 preferred_element_type=jnp.float32) + bb).reshape(B, NN, DIM))
    w1 = w1_r[...]
    acc = jnp.zeros((B, DIM), jnp.float32)
    for n in range(NN):
        acc = acc + w1[:, n][:, None] * h1[:, n, :]
    f = jnp.tanh(jnp.dot(h0 + acc, Wt, preferred_element_type=jnp.float32) + bb)
    out_r[...] = jax.nn.sigmoid(jnp.sum(u_r[...] * f, axis=-1))


def _tc_dense(u, e0, s0, agg1, w1, W, b):
    B = u.shape[0]
    return pl.pallas_call(
        _tc_body,
        out_shape=jax.ShapeDtypeStruct((B,), jnp.float32),
        compiler_params=pltpu.CompilerParams(
            vmem_limit_bytes=100 * 1024 * 1024),
    )(u, e0, s0, agg1, w1, W, b)


def kernel(usr_id, item_ids, adj_ent, adj_rel, usr_table, ent_table, rel_table, W, b):
    B = usr_id.shape[0]
    item_flat = item_ids.reshape(B).astype(jnp.int32)
    usr_flat = usr_id.reshape(B).astype(jnp.int32)
    adj_cat = jnp.concatenate(
        [adj_ent.astype(jnp.int32), adj_rel.astype(jnp.int32)], axis=1)

    u, e0, s0, agg1, w1 = _sc_fused(
        item_flat, usr_flat, adj_cat, ent_table, usr_table, rel_table)

    return _tc_dense(u, e0, s0, agg1, w1, W, b.reshape(1, DIM))


# speedup vs baseline: 1.9737x; 1.0110x over previous
import functools

import jax
import jax.numpy as jnp
from jax import lax
from jax.experimental import pallas as pl
from jax.experimental.pallas import tpu as pltpu
from jax.experimental.pallas import tpu_sc as plsc

NUM_REL = 32
DIM = 64
NN = 16


def _sc_fused(item_ids, usr_id, adj_cat, ent_table, usr_table, rel_table):
    B = item_ids.shape[0]
    info = plsc.get_sparse_core_info()
    NC, NS = info.num_cores, info.num_subcores
    NW = NC * NS
    bpw = B // NW

    mesh = plsc.VectorSubcoreMesh(core_axis_name="c", subcore_axis_name="s")

    out_type = (
        jax.ShapeDtypeStruct((B, DIM), jnp.float32),
        jax.ShapeDtypeStruct((B, DIM), jnp.float32),
        jax.ShapeDtypeStruct((B, DIM), jnp.float32),
        jax.ShapeDtypeStruct((B, NN, DIM), jnp.float32),
        jax.ShapeDtypeStruct((B, NN), jnp.float32),
    )
    f32, i32 = jnp.float32, jnp.int32
    scratch = [
        pltpu.VMEM((bpw,), i32),
        pltpu.VMEM((bpw,), i32),
        pltpu.VMEM((bpw, 2 * NN), i32),
        pltpu.VMEM((bpw,), i32),
        pltpu.VMEM((bpw,), i32),
        pltpu.VMEM((bpw,), i32),
        pltpu.VMEM((bpw, NN), i32),
        pltpu.VMEM((bpw, NN), i32),
        pltpu.VMEM((bpw, DIM), f32),
        pltpu.VMEM((bpw, DIM), f32),
        pltpu.VMEM((NUM_REL, DIM), f32),
        pltpu.VMEM((NUM_REL, bpw), f32),
        pltpu.VMEM((NN,), f32),
        [pltpu.VMEM((NN, 2 * NN), i32)] * 2,
        [pltpu.VMEM((2, 128), i32)] * 2,
        [pltpu.VMEM((NN, DIM), f32)] * 2,
        [pltpu.VMEM((NN * NN, DIM), f32)] * 2,
        [pltpu.VMEM((NN,), f32)] * 2,
        [pltpu.VMEM((DIM,), f32)] * 2,
        [pltpu.VMEM((NN, DIM), f32)] * 2,
        pltpu.SemaphoreType.DMA,
        [pltpu.SemaphoreType.DMA] * 2,
        [pltpu.SemaphoreType.DMA] * 2,
        [pltpu.SemaphoreType.DMA] * 2,
        [pltpu.SemaphoreType.DMA] * 2,
    ]

    @functools.partial(pl.kernel, out_type=out_type, mesh=mesh,
                       scratch_types=scratch,
                       compiler_params=pltpu.CompilerParams(
                           use_tc_tiling_on_sc=False,
                           needs_layout_passes=False))
    def k(item_h, usr_h, adj_h, ent_h, usrt_h, rel_h,
          u_o, e0_o, s0_o, agg1_o, w1_o,
          it_v, us_v, ar1_v, it4_v, it2_v, us2_v, arp4_v, arp2_v,
          e0_v, u_v, rel_v, p_v, ebuf,
          ar2_s, flat_s, e1_s, e2_s, w1buf_s, s0buf_s, aggbuf_s,
          sem_hdr, sem_a2, sem_er, sem_e2, sem_out):
        wid = lax.axis_index("s") * NC + lax.axis_index("c")
        base = wid * bpw
        iota16 = lax.iota(i32, NN)
        zeros16 = jnp.zeros((NN,), f32)

        def perm(e, sh):
            o = e & 4095
            return (e - o) | ((o << sh) & 4095) | lax.shift_right_logical(
                o, 12 - sh)

        pltpu.sync_copy(item_h.at[pl.ds(base, bpw)], it_v)
        pltpu.sync_copy(usr_h.at[pl.ds(base, bpw)], us_v)
        for g in range(bpw // NN):
            sl = pl.ds(g * NN, NN)
            it_g = it_v[sl]
            it4_v[sl] = perm(it_g, 2)
            it2_v[sl] = perm(it_g, 1)
            us2_v[sl] = perm(us_v[sl], 1)
        h1 = pltpu.async_copy(adj_h.at[it4_v], ar1_v, sem_hdr)
        h3 = pltpu.async_copy(ent_h.at[it2_v], e0_v, sem_hdr)
        h4 = pltpu.async_copy(usrt_h.at[us2_v], u_v, sem_hdr)
        pltpu.sync_copy(rel_h, rel_v)
        h1.wait(); h3.wait(); h4.wait()
        for i in range(bpw):
            a1row = ar1_v[i, pl.ds(0, NN)]
            arp4_v[i, pl.ds(0, NN)] = perm(a1row, 2)
            arp2_v[i, pl.ds(0, NN)] = perm(a1row, 1)
        pltpu.sync_copy(e0_v, e0_o.at[pl.ds(base, bpw)])
        pltpu.sync_copy(u_v, u_o.at[pl.ds(base, bpw)])

        for kk in range(NUM_REL):
            p_v[kk, pl.ds(0, NN)] = zeros16
            p_v[kk, pl.ds(NN, NN)] = zeros16

        def pbody(d, carry):
            dvec = jnp.full((NN,), d, i32)
            ucol0 = plsc.load_gather(u_v, [iota16, dvec])
            ucol1 = plsc.load_gather(u_v, [iota16 + NN, dvec])
            for kk in range(NUM_REL):
                wvec = plsc.load_gather(rel_v, [jnp.full((NN,), kk, i32), dvec])
                plsc.addupdate(p_v.at[kk, pl.ds(0, NN)], ucol0 * wvec)
                plsc.addupdate(p_v.at[kk, pl.ds(NN, NN)], ucol1 * wvec)
            return carry

        lax.fori_loop(0, DIM, pbody, 0)

        def fire_a2(i, s):
            return pltpu.async_copy(adj_h.at[arp4_v.at[i]], ar2_s[s], sem_a2[s])

        def fire_er(i, s):
            pltpu.async_copy(ent_h.at[arp2_v.at[i]], e1_s[s], sem_er[s])

        def wait_a2(i, s):
            pltpu.make_async_copy(adj_h.at[arp4_v.at[i]],
                                  ar2_s[s], sem_a2[s]).wait()

        def flatten_fire_e2(i, s):
            for j in range(NN):
                flat_s[s][j // 8, pl.ds((j % 8) * NN, NN)] = perm(
                    ar2_s[s][j, pl.ds(0, NN)], 1)
            pltpu.async_copy(ent_h.at[flat_s[s].at[0]],
                             e2_s[s].at[pl.ds(0, 128)], sem_e2[s])
            pltpu.async_copy(ent_h.at[flat_s[s].at[1]],
                             e2_s[s].at[pl.ds(128, 128)], sem_e2[s])

        def drain_outs(i, s):
            gbp = base + i
            pltpu.make_async_copy(w1buf_s[s], w1_o.at[gbp], sem_out[s]).wait()
            pltpu.make_async_copy(s0buf_s[s], s0_o.at[gbp], sem_out[s]).wait()
            pltpu.make_async_copy(aggbuf_s[s], agg1_o.at[gbp], sem_out[s]).wait()

        def compute(i, s):
            gb = base + i

            @pl.when(i >= 2)
            def _():
                drain_outs(i - 2, s)

            pltpu.make_async_copy(ent_h.at[arp2_v.at[i]],
                                  e1_s[s], sem_er[s]).wait()
            pltpu.make_async_copy(ent_h.at[flat_s[s].at[0]],
                                  e2_s[s].at[pl.ds(0, 128)], sem_e2[s]).wait()
            pltpu.make_async_copy(ent_h.at[flat_s[s].at[1]],
                                  e2_s[s].at[pl.ds(128, 128)], sem_e2[s]).wait()

            ivec = jnp.full((NN,), i, i32)
            r1vec = plsc.load_gather(ar1_v, [ivec, iota16 + NN])
            sc1 = plsc.load_gather(p_v, [r1vec, ivec])
            es1 = jnp.exp(sc1)
            w1vec = es1 / jnp.sum(es1)
            w1buf_s[s][...] = w1vec
            acc0 = [zeros16] * 4
            for n in range(NN):
                w = plsc.load_gather(w1buf_s[s], [jnp.full((NN,), n, i32)])
                for c in range(4):
                    acc0[c] = acc0[c] + w * e1_s[s][n, pl.ds(16 * c, 16)]
            for c in range(4):
                s0buf_s[s][pl.ds(16 * c, 16)] = acc0[c]

            for m in range(NN):
                r2vec = ar2_s[s][m, pl.ds(NN, NN)]
                sc2 = plsc.load_gather(p_v, [r2vec, ivec])
                es2 = jnp.exp(sc2)
                ssum2 = jnp.sum(es2)
                ebuf[...] = es2
                acca = [zeros16] * 4
                accb = [zeros16] * 4
                for n in range(0, NN, 2):
                    wa = plsc.load_gather(ebuf, [jnp.full((NN,), n, i32)])
                    wb = plsc.load_gather(ebuf, [jnp.full((NN,), n + 1, i32)])
                    rowa = m * NN + n
                    rowb = rowa + 1
                    for c in range(4):
                        acca[c] = acca[c] + wa * e2_s[s][rowa, pl.ds(16 * c, 16)]
                        accb[c] = accb[c] + wb * e2_s[s][rowb, pl.ds(16 * c, 16)]
                for c in range(4):
                    aggbuf_s[s][m, pl.ds(16 * c, 16)] = (
                        e1_s[s][m, pl.ds(16 * c, 16)] + (acca[c] + accb[c]) / ssum2)

            pltpu.async_copy(w1buf_s[s], w1_o.at[gb], sem_out[s])
            pltpu.async_copy(s0buf_s[s], s0_o.at[gb], sem_out[s])
            pltpu.async_copy(aggbuf_s[s], agg1_o.at[gb], sem_out[s])

        fire_a2(0, 0)
        fire_er(0, 0)
        wait_a2(0, 0)
        flatten_fire_e2(0, 0)
        fire_a2(1, 1)
        fire_er(1, 1)

        def body(j, carry):
            r0 = 2 * j
            more = j < (bpw // 2 - 1)

            @pl.when(more)
            def _():
                fire_a2(r0 + 2, 0)

            wait_a2(r0 + 1, 1)
            flatten_fire_e2(r0 + 1, 1)
            compute(r0, 0)

            @pl.when(more)
            def _():
                fire_er(r0 + 2, 0)
                wait_a2(r0 + 2, 0)
                flatten_fire_e2(r0 + 2, 0)
                fire_a2(r0 + 3, 1)

            compute(r0 + 1, 1)

            @pl.when(more)
            def _():
                fire_er(r0 + 3, 1)

            return carry

        lax.fori_loop(0, bpw // 2, body, 0)

        drain_outs(bpw - 2, 0)
        drain_outs(bpw - 1, 1)

    return k(item_ids, usr_id, adj_cat, ent_table, usr_table, rel_table)


def _tc_linearize(x, width, dtype):
    xt = jnp.swapaxes(x, 0, 1)
    n = x.shape[0]
    NP = 102400
    xt = jnp.pad(xt, ((0, 0), (0, NP - n)))
    CH = 4096
    per = CH * width // 128
    grid = (NP // CH,)

    k = 128 // width
    sub = CH // k

    def body(in_r, out_r):
        v = in_r[...].T
        parts = [v[j * sub:(j + 1) * sub] for j in range(k)]
        out_r[...] = jnp.concatenate(parts, axis=1)

    out = pl.pallas_call(
        body,
        grid=grid,
        in_specs=[pl.BlockSpec((width, CH), lambda i: (0, i))],
        out_specs=pl.BlockSpec((per, 128), lambda i: (i, 0)),
        out_shape=jax.ShapeDtypeStruct((NP * width // 128, 128), dtype),
        compiler_params=pltpu.CompilerParams(
            vmem_limit_bytes=100 * 1024 * 1024),
    )(xt)
    return out.reshape(NP, width)


def _tc_body(u_r, e0_r, s0_r, agg_r, w1_r, W_r, b_r, out_r):
    B = u_r.shape[0]
    Wt = W_r[...].T
    bb = b_r[...]
    h0 = jax.nn.sigmoid(
        jnp.dot(e0_r[...] + s0_r[...], Wt, preferred_element_type=jnp.float32) + bb)
    agg = agg_r[...]
    h1 = jax.nn.sigmoid(
        (jnp.dot(agg.reshape(B * NN, DIM), Wt,
                 preferred_element_type=jnp.float32) + bb).reshape(B, NN, DIM))
    w1 = w1_r[...]
    acc = jnp.zeros((B, DIM), jnp.float32)
    for n in range(NN):
        acc = acc + w1[:, n][:, None] * h1[:, n, :]
    f = jnp.tanh(jnp.dot(h0 + acc, Wt, preferred_element_type=jnp.float32) + bb)
    out_r[...] = jax.nn.sigmoid(jnp.sum(u_r[...] * f, axis=-1))


def _tc_dense(u, e0, s0, agg1, w1, W, b):
    B = u.shape[0]
    return pl.pallas_call(
        _tc_body,
        out_shape=jax.ShapeDtypeStruct((B,), jnp.float32),
        compiler_params=pltpu.CompilerParams(
            vmem_limit_bytes=100 * 1024 * 1024),
    )(u, e0, s0, agg1, w1, W, b)


def kernel(usr_id, item_ids, adj_ent, adj_rel, usr_table, ent_table, rel_table, W, b):
    B = usr_id.shape[0]
    item_flat = item_ids.reshape(B).astype(jnp.int32)
    usr_flat = usr_id.reshape(B).astype(jnp.int32)
    adj_cat = jnp.concatenate(
        [adj_ent.astype(jnp.int32), adj_rel.astype(jnp.int32)], axis=1)
    adj_lin = _tc_linearize(adj_cat, 2 * NN, jnp.int32)
    ent_lin = _tc_linearize(ent_table, DIM, jnp.float32)
    usr_lin = _tc_linearize(usr_table, DIM, jnp.float32)

    u, e0, s0, agg1, w1 = _sc_fused(
        item_flat, usr_flat, adj_lin, ent_lin, usr_lin, rel_table)

    return _tc_dense(u, e0, s0, agg1, w1, W, b.reshape(1, DIM))


# speedup vs baseline: 2.3145x; 1.1727x over previous
import functools

import jax
import jax.numpy as jnp
from jax import lax
from jax.experimental import pallas as pl
from jax.experimental.pallas import tpu as pltpu
from jax.experimental.pallas import tpu_sc as plsc

NUM_REL = 32
DIM = 64
NN = 16


def _sc_fused(item_ids, usr_id, adj_cat, ent_table, usr_table, rel_table):
    B = item_ids.shape[0]
    info = plsc.get_sparse_core_info()
    NC, NS = info.num_cores, info.num_subcores
    NW = NC * NS
    bpw = B // NW

    mesh = plsc.VectorSubcoreMesh(core_axis_name="c", subcore_axis_name="s")

    out_type = (
        jax.ShapeDtypeStruct((B, DIM), jnp.float32),
        jax.ShapeDtypeStruct((B, DIM), jnp.float32),
        jax.ShapeDtypeStruct((B, DIM), jnp.float32),
        jax.ShapeDtypeStruct((B, NN, DIM), jnp.float32),
        jax.ShapeDtypeStruct((B, NN), jnp.float32),
    )
    f32, i32 = jnp.float32, jnp.int32
    scratch = [
        pltpu.VMEM((bpw,), i32),
        pltpu.VMEM((bpw,), i32),
        pltpu.VMEM((bpw, 2 * NN), i32),
        pltpu.VMEM((bpw,), i32),
        pltpu.VMEM((bpw,), i32),
        pltpu.VMEM((bpw,), i32),
        pltpu.VMEM((bpw, NN), i32),
        pltpu.VMEM((bpw, NN), i32),
        pltpu.VMEM((bpw, DIM), f32),
        pltpu.VMEM((bpw, DIM), f32),
        pltpu.VMEM((NUM_REL, DIM), f32),
        pltpu.VMEM((NUM_REL, bpw), f32),
        pltpu.VMEM((NN,), f32),
        [pltpu.VMEM((NN, 2 * NN), i32)] * 2,
        [pltpu.VMEM((2, 128), i32)] * 2,
        [pltpu.VMEM((NN, DIM), f32)] * 2,
        [pltpu.VMEM((NN * NN, DIM), f32)] * 2,
        [pltpu.VMEM((NN,), f32)] * 2,
        [pltpu.VMEM((DIM,), f32)] * 2,
        [pltpu.VMEM((NN, DIM), f32)] * 2,
        pltpu.SemaphoreType.DMA,
        [pltpu.SemaphoreType.DMA] * 2,
        [pltpu.SemaphoreType.DMA] * 2,
        [pltpu.SemaphoreType.DMA] * 2,
        [pltpu.SemaphoreType.DMA] * 2,
    ]

    @functools.partial(pl.kernel, out_type=out_type, mesh=mesh,
                       scratch_types=scratch,
                       compiler_params=pltpu.CompilerParams(
                           use_tc_tiling_on_sc=False,
                           needs_layout_passes=False))
    def k(item_h, usr_h, adj_h, ent_h, usrt_h, rel_h,
          u_o, e0_o, s0_o, agg1_o, w1_o,
          it_v, us_v, ar1_v, it4_v, it2_v, us2_v, arp4_v, arp2_v,
          e0_v, u_v, rel_v, p_v, ebuf,
          ar2_s, flat_s, e1_s, e2_s, w1buf_s, s0buf_s, aggbuf_s,
          sem_hdr, sem_a2, sem_er, sem_e2, sem_out):
        wid = lax.axis_index("s") * NC + lax.axis_index("c")
        base = wid * bpw
        iota16 = lax.iota(i32, NN)
        zeros16 = jnp.zeros((NN,), f32)

        def perm(e, sh):
            o = e & 4095
            return (e - o) | ((o << sh) & 4095) | lax.shift_right_logical(
                o, 12 - sh)

        pltpu.sync_copy(item_h.at[pl.ds(base, bpw)], it_v)
        pltpu.sync_copy(usr_h.at[pl.ds(base, bpw)], us_v)
        for g in range(bpw // NN):
            sl = pl.ds(g * NN, NN)
            it_g = it_v[sl]
            it4_v[sl] = perm(it_g, 2)
            it2_v[sl] = perm(it_g, 1)
            us2_v[sl] = perm(us_v[sl], 1)
        h1 = pltpu.async_copy(adj_h.at[it4_v], ar1_v, sem_hdr)
        h3 = pltpu.async_copy(ent_h.at[it2_v], e0_v, sem_hdr)
        h4 = pltpu.async_copy(usrt_h.at[us2_v], u_v, sem_hdr)
        pltpu.sync_copy(rel_h, rel_v)
        h1.wait(); h3.wait(); h4.wait()
        for i in range(bpw):
            a1row = ar1_v[i, pl.ds(0, NN)]
            arp4_v[i, pl.ds(0, NN)] = perm(a1row, 2)
            arp2_v[i, pl.ds(0, NN)] = perm(a1row, 1)
        pltpu.sync_copy(e0_v, e0_o.at[pl.ds(base, bpw)])
        pltpu.sync_copy(u_v, u_o.at[pl.ds(base, bpw)])

        for kk in range(NUM_REL):
            p_v[kk, pl.ds(0, NN)] = zeros16
            p_v[kk, pl.ds(NN, NN)] = zeros16

        def pbody(d, carry):
            dvec = jnp.full((NN,), d, i32)
            ucol0 = plsc.load_gather(u_v, [iota16, dvec])
            ucol1 = plsc.load_gather(u_v, [iota16 + NN, dvec])
            for kk in range(NUM_REL):
                wvec = plsc.load_gather(rel_v, [jnp.full((NN,), kk, i32), dvec])
                plsc.addupdate(p_v.at[kk, pl.ds(0, NN)], ucol0 * wvec)
                plsc.addupdate(p_v.at[kk, pl.ds(NN, NN)], ucol1 * wvec)
            return carry

        lax.fori_loop(0, DIM, pbody, 0)

        def fire_a2(i, s):
            return pltpu.async_copy(adj_h.at[arp4_v.at[i]], ar2_s[s], sem_a2[s])

        def fire_er(i, s):
            pltpu.async_copy(ent_h.at[arp2_v.at[i]], e1_s[s], sem_er[s])

        def wait_a2(i, s):
            pltpu.make_async_copy(adj_h.at[arp4_v.at[i]],
                                  ar2_s[s], sem_a2[s]).wait()

        def flatten_fire_e2(i, s):
            for j in range(NN):
                flat_s[s][j // 8, pl.ds((j % 8) * NN, NN)] = perm(
                    ar2_s[s][j, pl.ds(0, NN)], 1)
            pltpu.async_copy(ent_h.at[flat_s[s].at[0]],
                             e2_s[s].at[pl.ds(0, 128)], sem_e2[s])
            pltpu.async_copy(ent_h.at[flat_s[s].at[1]],
                             e2_s[s].at[pl.ds(128, 128)], sem_e2[s])

        def drain_outs(i, s):
            gbp = base + i
            pltpu.make_async_copy(w1buf_s[s], w1_o.at[gbp], sem_out[s]).wait()
            pltpu.make_async_copy(s0buf_s[s], s0_o.at[gbp], sem_out[s]).wait()
            pltpu.make_async_copy(aggbuf_s[s], agg1_o.at[gbp], sem_out[s]).wait()

        def compute(i, s):
            gb = base + i

            @pl.when(i >= 2)
            def _():
                drain_outs(i - 2, s)

            pltpu.make_async_copy(ent_h.at[arp2_v.at[i]],
                                  e1_s[s], sem_er[s]).wait()
            pltpu.make_async_copy(ent_h.at[flat_s[s].at[0]],
                                  e2_s[s].at[pl.ds(0, 128)], sem_e2[s]).wait()
            pltpu.make_async_copy(ent_h.at[flat_s[s].at[1]],
                                  e2_s[s].at[pl.ds(128, 128)], sem_e2[s]).wait()

            ivec = jnp.full((NN,), i, i32)
            r1vec = plsc.load_gather(ar1_v, [ivec, iota16 + NN])
            sc1 = plsc.load_gather(p_v, [r1vec, ivec])
            es1 = jnp.exp(sc1)
            w1vec = es1 / jnp.sum(es1)
            w1buf_s[s][...] = w1vec
            acc0 = [zeros16] * 4
            for n in range(NN):
                w = plsc.load_gather(w1buf_s[s], [jnp.full((NN,), n, i32)])
                for c in range(4):
                    acc0[c] = acc0[c] + w * e1_s[s][n, pl.ds(16 * c, 16)]
            for c in range(4):
                s0buf_s[s][pl.ds(16 * c, 16)] = acc0[c]

            for m in range(NN):
                r2vec = ar2_s[s][m, pl.ds(NN, NN)]
                sc2 = plsc.load_gather(p_v, [r2vec, ivec])
                es2 = jnp.exp(sc2)
                ssum2 = jnp.sum(es2)
                ebuf[...] = es2
                acca = [zeros16] * 4
                accb = [zeros16] * 4
                for n in range(0, NN, 2):
                    wa = plsc.load_gather(ebuf, [jnp.full((NN,), n, i32)])
                    wb = plsc.load_gather(ebuf, [jnp.full((NN,), n + 1, i32)])
                    rowa = m * NN + n
                    rowb = rowa + 1
                    for c in range(4):
                        acca[c] = acca[c] + wa * e2_s[s][rowa, pl.ds(16 * c, 16)]
                        accb[c] = accb[c] + wb * e2_s[s][rowb, pl.ds(16 * c, 16)]
                for c in range(4):
                    aggbuf_s[s][m, pl.ds(16 * c, 16)] = (
                        e1_s[s][m, pl.ds(16 * c, 16)] + (acca[c] + accb[c]) / ssum2)

            pltpu.async_copy(w1buf_s[s], w1_o.at[gb], sem_out[s])
            pltpu.async_copy(s0buf_s[s], s0_o.at[gb], sem_out[s])
            pltpu.async_copy(aggbuf_s[s], agg1_o.at[gb], sem_out[s])

        fire_a2(0, 0)
        fire_er(0, 0)
        wait_a2(0, 0)
        flatten_fire_e2(0, 0)
        fire_a2(1, 1)
        fire_er(1, 1)

        def body(j, carry):
            r0 = 2 * j
            more = j < (bpw // 2 - 1)

            @pl.when(more)
            def _():
                fire_a2(r0 + 2, 0)

            wait_a2(r0 + 1, 1)
            flatten_fire_e2(r0 + 1, 1)
            compute(r0, 0)

            @pl.when(more)
            def _():
                fire_er(r0 + 2, 0)
                wait_a2(r0 + 2, 0)
                flatten_fire_e2(r0 + 2, 0)
                fire_a2(r0 + 3, 1)

            compute(r0 + 1, 1)

            @pl.when(more)
            def _():
                fire_er(r0 + 3, 1)

            return carry

        lax.fori_loop(0, bpw // 2, body, 0)

        drain_outs(bpw - 2, 0)
        drain_outs(bpw - 1, 1)

    return k(item_ids, usr_id, adj_cat, ent_table, usr_table, rel_table)


def _tc_linearize(x, width, dtype, use_mxu):
    xt = jnp.swapaxes(x, 0, 1)
    NP = 102400
    CH = 4096
    per = CH * width // 128
    grid = (NP // CH,)

    k = 128 // width
    sub = CH // k

    def body(in_r, out_r):
        v = in_r[...]
        if use_mxu:
            ii = jax.lax.broadcasted_iota(jnp.int32, (width, width), 0)
            jj = jax.lax.broadcasted_iota(jnp.int32, (width, width), 1)
            eye = (ii == jj).astype(jnp.float32)
            vt = jax.lax.dot_general(v, eye, (((0,), (0,)), ((), ())),
                                     preferred_element_type=jnp.float32)
        else:
            vt = v.T
        parts = [vt[j * sub:(j + 1) * sub] for j in range(k)]
        out_r[...] = jnp.concatenate(parts, axis=1)

    out = pl.pallas_call(
        body,
        grid=grid,
        in_specs=[pl.BlockSpec((width, CH), lambda i: (0, i))],
        out_specs=pl.BlockSpec((per, 128), lambda i: (i, 0)),
        out_shape=jax.ShapeDtypeStruct((NP * width // 128, 128), dtype),
        compiler_params=pltpu.CompilerParams(
            vmem_limit_bytes=100 * 1024 * 1024),
    )(xt)
    return out.reshape(NP, width)


def _tc_body(u_r, e0_r, s0_r, agg_r, w1_r, W_r, b_r, out_r):
    B = u_r.shape[0]
    Wt = W_r[...].T
    bb = b_r[...]
    h0 = jax.nn.sigmoid(
        jnp.dot(e0_r[...] + s0_r[...], Wt, preferred_element_type=jnp.float32) + bb)
    agg = agg_r[...]
    h1 = jax.nn.sigmoid(
        (jnp.dot(agg.reshape(B * NN, DIM), Wt,
                 preferred_element_type=jnp.float32) + bb).reshape(B, NN, DIM))
    w1 = w1_r[...]
    acc = jnp.zeros((B, DIM), jnp.float32)
    for n in range(NN):
        acc = acc + w1[:, n][:, None] * h1[:, n, :]
    f = jnp.tanh(jnp.dot(h0 + acc, Wt, preferred_element_type=jnp.float32) + bb)
    out_r[...] = jax.nn.sigmoid(jnp.sum(u_r[...] * f, axis=-1))


def _tc_dense(u, e0, s0, agg1, w1, W, b):
    B = u.shape[0]
    return pl.pallas_call(
        _tc_body,
        out_shape=jax.ShapeDtypeStruct((B,), jnp.float32),
        compiler_params=pltpu.CompilerParams(
            vmem_limit_bytes=100 * 1024 * 1024),
    )(u, e0, s0, agg1, w1, W, b)


def kernel(usr_id, item_ids, adj_ent, adj_rel, usr_table, ent_table, rel_table, W, b):
    B = usr_id.shape[0]
    item_flat = item_ids.reshape(B).astype(jnp.int32)
    usr_flat = usr_id.reshape(B).astype(jnp.int32)
    adj_cat = jnp.concatenate(
        [adj_ent.astype(jnp.int32), adj_rel.astype(jnp.int32)], axis=1)
    adj_lin = _tc_linearize(adj_cat, 2 * NN, jnp.int32, use_mxu=False)
    ent_lin = _tc_linearize(ent_table, DIM, jnp.float32, use_mxu=True)
    usr_lin = _tc_linearize(usr_table, DIM, jnp.float32, use_mxu=True)

    u, e0, s0, agg1, w1 = _sc_fused(
        item_flat, usr_flat, adj_lin, ent_lin, usr_lin, rel_table)

    return _tc_dense(u, e0, s0, agg1, w1, W, b.reshape(1, DIM))
